# baseline (speedup 1.0000x reference)
import jax
import jax.numpy as jnp
from jax import lax
from jax.experimental import pallas as pl
from jax.experimental.pallas import tpu as pltpu
from jax.experimental.pallas import tpu_sc as plsc

B = 1024
SIZE = 26
D = 4096
NUM_LEVELS = 100
LANES = 16
NSUB = 16
NTILES = 32
COLS = D // NTILES
CHUNK = 4
NCHUNK = B // CHUNK
CVR = COLS // LANES


def _sc_kernel(x_ref, pos_ref, lev_ref, out_ref,
               lev_sh, w_v, psum_v, x_v, idx_v, rows_v, out_v, sem):
    cid = lax.axis_index("c")
    sid = lax.axis_index("s")
    col0 = (cid * NSUB + sid) * COLS

    pltpu.sync_copy(lev_ref.at[:, pl.ds(col0, COLS)],
                    lev_sh.at[pl.ds(sid * NUM_LEVELS, NUM_LEVELS), :])
    pltpu.sync_copy(pos_ref.at[:, pl.ds(col0, COLS)], w_v)
    pltpu.sync_copy(x_ref, x_v)
    plsc.subcore_barrier()

    for c in range(CVR):
        acc = jnp.zeros((LANES,), jnp.float32)
        for s in range(SIZE):
            acc = acc + w_v[s, pl.ds(c * LANES, LANES)]
        psum_v[pl.ds(c * LANES, LANES)] = acc

    def _flip(s, _):
        for c in range(CVR):
            p = w_v[s, pl.ds(c * LANES, LANES)]
            w_v[s, pl.ds(c * LANES, LANES)] = 1.0 - 2.0 * p
        return ()
    lax.fori_loop(0, SIZE, _flip, (), unroll=False)

    row0 = sid * NUM_LEVELS

    def _quant(i, _):
        v = x_v[pl.ds(i * LANES, LANES)] * float(NUM_LEVELS)
        vi = v.astype(jnp.int32)
        vi = jnp.minimum(jnp.maximum(vi, 0), NUM_LEVELS - 1) + row0
        idx_v[pl.ds(i * LANES, LANES)] = vi
        return ()
    lax.fori_loop(0, (B * SIZE) // LANES, _quant, (), unroll=False)

    def _chunk(g, _):
        idxc = idx_v.at[pl.ds(g * (CHUNK * SIZE), CHUNK * SIZE)]
        pltpu.async_copy(lev_sh.at[idxc], rows_v, sem).wait()

        for b in range(CHUNK):
            for c in range(CVR):
                out_v[b, pl.ds(c * LANES, LANES)] = psum_v[
                    pl.ds(c * LANES, LANES)]

        def _slot(s, _):
            for b in range(CHUNK):
                for c in range(CVR):
                    r = rows_v[b * SIZE + s, pl.ds(c * LANES, LANES)]
                    w = w_v[s, pl.ds(c * LANES, LANES)]
                    plsc.addupdate(out_v.at[b, pl.ds(c * LANES, LANES)],
                                   r * w)
            return ()
        lax.fori_loop(0, SIZE, _slot, (), unroll=False)

        pltpu.sync_copy(out_v,
                        out_ref.at[pl.ds(g * CHUNK, CHUNK),
                                   pl.ds(col0, COLS)])
        return ()
    lax.fori_loop(0, NCHUNK, _chunk, (), unroll=False)


@jax.jit
def kernel(x, position, levels):
    run = pl.kernel(
        _sc_kernel,
        mesh=plsc.VectorSubcoreMesh(core_axis_name="c", subcore_axis_name="s"),
        out_type=jax.ShapeDtypeStruct((B, D), jnp.float32),
        scratch_types=[
            pltpu.VMEM_SHARED((NSUB * NUM_LEVELS, COLS), jnp.float32),
            pltpu.VMEM((SIZE, COLS), jnp.float32),
            pltpu.VMEM((COLS,), jnp.float32),
            pltpu.VMEM((B * SIZE,), jnp.float32),
            pltpu.VMEM((B * SIZE,), jnp.int32),
            pltpu.VMEM((CHUNK * SIZE, COLS), jnp.float32),
            pltpu.VMEM((CHUNK, COLS), jnp.float32),
            pltpu.SemaphoreType.DMA,
        ],
    )
    return run(x.reshape(B * SIZE), position, levels)


# speedup vs baseline: 31.9833x; 31.9833x over previous
import jax
import jax.numpy as jnp
from jax.experimental import pallas as pl
from jax.experimental.pallas import tpu as pltpu

B = 1024
SIZE = 26
D = 4096
NUM_LEVELS = 100
B_TILE = 256
K = SIZE * 128


def _encode_kernel(x_ref, pos_ref, lev_ref, out_ref, tab_ref):
    @pl.when(pl.program_id(0) == 0)
    def _build_table():
        lev = jnp.concatenate(
            [lev_ref[...].astype(jnp.bfloat16),
             jnp.zeros((128 - NUM_LEVELS, D), jnp.bfloat16)], axis=0)
        for s in range(SIZE):
            w = pos_ref[s, :][None, :].astype(jnp.bfloat16)
            tab_ref[pl.ds(s * 128, 128), :] = lev + w - 2.0 * lev * w

    idx = jnp.clip(jnp.floor(x_ref[...] * NUM_LEVELS), 0, NUM_LEVELS - 1)
    idx = idx.astype(jnp.int32)
    iota = jax.lax.broadcasted_iota(jnp.int32, (1, 128), 1)
    hots = []
    for s in range(SIZE):
        hots.append((idx[:, s][:, None] == iota).astype(jnp.bfloat16))
    onehot = jnp.concatenate(hots, axis=1)
    out_ref[...] = jnp.dot(onehot, tab_ref[...],
                           preferred_element_type=jnp.float32)


@jax.jit
def kernel(x, position, levels):
    return pl.pallas_call(
        _encode_kernel,
        grid=(B // B_TILE,),
        in_specs=[
            pl.BlockSpec((B_TILE, SIZE), lambda j: (j, 0)),
            pl.BlockSpec((SIZE, D), lambda j: (0, 0)),
            pl.BlockSpec((NUM_LEVELS, D), lambda j: (0, 0)),
        ],
        out_specs=pl.BlockSpec((B_TILE, D), lambda j: (j, 0)),
        out_shape=jax.ShapeDtypeStruct((B, D), jnp.float32),
        scratch_shapes=[pltpu.VMEM((K, D), jnp.bfloat16)],
    )(x, position, levels)


# speedup vs baseline: 36.0035x; 1.1257x over previous
import jax
import jax.numpy as jnp
from jax.experimental import pallas as pl

B = 1024
SIZE = 26
D = 4096
NUM_LEVELS = 100
D_TILE = 512


def _encode_kernel(x_ref, pos_ref, lev_ref, out_ref):
    idx = jnp.clip(jnp.floor(x_ref[...] * NUM_LEVELS), 0, NUM_LEVELS - 1)
    idx = idx.astype(jnp.int32)
    lev = lev_ref[...]
    iota = jax.lax.broadcasted_iota(jnp.int32, (1, 128), 1)
    zpad = jnp.zeros((128 - NUM_LEVELS, D_TILE), jnp.bfloat16)
    tabs, hots = [], []
    for s in range(SIZE):
        w = pos_ref[s, :][None, :]
        m_s = (lev + w - 2.0 * lev * w).astype(jnp.bfloat16)
        tabs.append(jnp.concatenate([m_s, zpad], axis=0))
        hots.append((idx[:, s][:, None] == iota).astype(jnp.bfloat16))
    table = jnp.concatenate(tabs, axis=0)
    onehot = jnp.concatenate(hots, axis=1)
    out_ref[...] = jnp.dot(onehot, table, preferred_element_type=jnp.float32)


@jax.jit
def kernel(x, position, levels):
    grid = (D // D_TILE,)
    return pl.pallas_call(
        _encode_kernel,
        grid=grid,
        in_specs=[
            pl.BlockSpec((B, SIZE), lambda j: (0, 0)),
            pl.BlockSpec((SIZE, D_TILE), lambda j: (0, j)),
            pl.BlockSpec((NUM_LEVELS, D_TILE), lambda j: (0, j)),
        ],
        out_specs=pl.BlockSpec((B, D_TILE), lambda j: (0, j)),
        out_shape=jax.ShapeDtypeStruct((B, D), jnp.float32),
    )(x, position, levels)
